# baseline (speedup 1.0000x reference)
import functools

import jax
import jax.numpy as jnp
from jax import lax
from jax.experimental import pallas as pl
from jax.experimental.pallas import tpu as pltpu
from jax.experimental.pallas import tpu_sc as plsc

NUM_MASKS = 1024
H = 224
W = 224
D = H * W
N = 1024

NUM_CORES = 2
NUM_SUBCORES = 16
NUM_WORKERS = NUM_CORES * NUM_SUBCORES
ROWS_PER = N // NUM_WORKERS
SCALE = 1.0 / 255.0


def _sampler_body(masks_hbm, ids8_hbm, out_hbm, idx_pad, buf0, buf1,
                  sem0, sem1):
    wid = lax.axis_index("s") * NUM_CORES + lax.axis_index("c")
    base = wid * ROWS_PER

    pltpu.sync_copy(ids8_hbm.at[pl.ds(base * 8, ROWS_PER * 8)], idx_pad)

    def gather(j, buf, sem):
        return pltpu.make_async_copy(
            masks_hbm.at[idx_pad.at[pl.ds(pl.multiple_of(j * 8, 8), 1)]],
            buf, sem)

    gather(0, buf0, sem0).start()
    gather(1, buf1, sem1).start()

    def process(j, buf, sem):
        gather(j, buf, sem).wait()
        row = buf.at[0]

        @plsc.parallel_loop(0, D, 16, unroll=8)
        def _scale(k):
            row[pl.ds(k, 16)] = row[pl.ds(k, 16)] * SCALE

        pltpu.sync_copy(buf, out_hbm.at[pl.ds(base + j, 1)])

        @pl.when(j + 2 < ROWS_PER)
        def _():
            gather(j + 2, buf, sem).start()

    def outer(t, carry):
        process(2 * t, buf0, sem0)
        process(2 * t + 1, buf1, sem1)
        return carry

    lax.fori_loop(0, ROWS_PER // 2, outer, 0)


@jax.jit
def _sampler(masks_flat, ids):
    mesh = plsc.VectorSubcoreMesh(core_axis_name="c", subcore_axis_name="s")
    run = functools.partial(
        pl.kernel,
        out_type=jax.ShapeDtypeStruct((N, D), jnp.float32),
        mesh=mesh,
        scratch_types=[
            pltpu.VMEM((ROWS_PER * 8,), jnp.int32),
            pltpu.VMEM((1, D), jnp.float32),
            pltpu.VMEM((1, D), jnp.float32),
            pltpu.SemaphoreType.DMA,
            pltpu.SemaphoreType.DMA,
        ],
    )(_sampler_body)
    return run(masks_flat, ids)


def kernel(masks, rand_id):
    masks_flat = masks.reshape(NUM_MASKS, D)
    ids8 = jnp.repeat(rand_id.astype(jnp.int32), 8)
    out = _sampler(masks_flat, ids8)
    return out.reshape(N, 1, H, W)


# speedup vs baseline: 3.2978x; 3.2978x over previous
import functools

import jax
import jax.numpy as jnp
from jax import lax
from jax.experimental import pallas as pl
from jax.experimental.pallas import tpu as pltpu
from jax.experimental.pallas import tpu_sc as plsc

NUM_MASKS = 1024
H = 224
W = 224
N = 1024
P = H * W
NBLK = P // 8
CHUNK = 8 * N
TOTAL = P * N

NUM_CORES = 2
NUM_SUBCORES = 16
NUM_WORKERS = NUM_CORES * NUM_SUBCORES
BLK_PER_W = NBLK // NUM_WORKERS
SCALE = 1.0 / 255.0


def _sampler_body(in_hbm, ids_hbm, out_hbm, idv, in0, in1, ob0, ob1,
                  si0, si1, so0, so1):
    wid = lax.axis_index("s") * NUM_CORES + lax.axis_index("c")
    b0 = wid * BLK_PER_W

    pltpu.sync_copy(ids_hbm, idv)

    def gin(c, buf, sem):
        return pltpu.make_async_copy(
            in_hbm.at[pl.ds((b0 + c) * CHUNK, CHUNK)], buf, sem)

    def gout(c, buf, sem):
        return pltpu.make_async_copy(
            buf, out_hbm.at[pl.ds((b0 + c) * CHUNK, CHUNK)], sem)

    gin(0, in0, si0).start()
    gin(1, in1, si1).start()

    def process(c, ibuf, obuf, sin, sout):
        gin(c, ibuf, sin).wait()

        @pl.when(c >= 2)
        def _():
            gout(c - 2, obuf, sout).wait()

        @plsc.parallel_loop(0, 64, 1)
        def _t(t):
            nb = t >> 3
            g = t & 7
            src = pl.multiple_of(t * 16, 16)
            idxv = idv[pl.ds(src, 16)]
            obase = nb * 1024 + g * 16
            for ws in range(8):
                v = plsc.load_gather(ibuf, [idxv + ws * 1024])
                dst = pl.multiple_of(obase + ws * 128, 16)
                obuf[pl.ds(dst, 16)] = v * SCALE

        gout(c, obuf, sout).start()

        @pl.when(c + 2 < BLK_PER_W)
        def _():
            gin(c + 2, ibuf, sin).start()

    def outer(t2, carry):
        process(2 * t2, in0, ob0, si0, so0)
        process(2 * t2 + 1, in1, ob1, si1, so1)
        return carry

    lax.fori_loop(0, BLK_PER_W // 2, outer, 0)
    gout(BLK_PER_W - 2, ob0, so0).wait()
    gout(BLK_PER_W - 1, ob1, so1).wait()


@jax.jit
def _sampler(flat_in, ids):
    mesh = plsc.VectorSubcoreMesh(core_axis_name="c", subcore_axis_name="s")
    run = functools.partial(
        pl.kernel,
        out_type=jax.ShapeDtypeStruct((TOTAL,), jnp.float32),
        mesh=mesh,
        compiler_params=pltpu.CompilerParams(needs_layout_passes=False),
        scratch_types=[
            pltpu.VMEM((N,), jnp.int32),
            pltpu.VMEM((CHUNK,), jnp.float32),
            pltpu.VMEM((CHUNK,), jnp.float32),
            pltpu.VMEM((CHUNK,), jnp.float32),
            pltpu.VMEM((CHUNK,), jnp.float32),
            pltpu.SemaphoreType.DMA,
            pltpu.SemaphoreType.DMA,
            pltpu.SemaphoreType.DMA,
            pltpu.SemaphoreType.DMA,
        ],
    )(_sampler_body)
    return run(flat_in, ids)


def kernel(masks, rand_id):
    flat_in = jnp.transpose(masks, (1, 2, 3, 0)).reshape(TOTAL)
    ids = rand_id.astype(jnp.int32)
    out1d = _sampler(flat_in, ids)
    out5 = out1d.reshape(H, W // 8, 8, 8, 128)
    out = jnp.transpose(out5, (2, 4, 0, 1, 3)).reshape(N, H, W)
    return out[:, None, :, :]


# speedup vs baseline: 3.9769x; 1.2059x over previous
import functools

import jax
import jax.numpy as jnp
from jax import lax
from jax.experimental import pallas as pl
from jax.experimental.pallas import tpu as pltpu
from jax.experimental.pallas import tpu_sc as plsc

NUM_MASKS = 1024
H = 224
W = 224
N = 1024
P = H * W
ROWS = 16
NBLK = P // ROWS
CHUNK = ROWS * N
TOTAL = P * N

NUM_CORES = 2
NUM_SUBCORES = 16
NUM_WORKERS = NUM_CORES * NUM_SUBCORES
BLK_PER_W = NBLK // NUM_WORKERS
SCALE = 1.0 / 255.0


def _sampler_body(in_hbm, ids_hbm, out_hbm, idv, in0, in1, ob0, ob1,
                  si0, si1, so0, so1):
    wid = lax.axis_index("s") * NUM_CORES + lax.axis_index("c")
    b0 = wid * BLK_PER_W

    pltpu.sync_copy(ids_hbm, idv)

    def gin(c, buf, sem):
        return pltpu.make_async_copy(
            in_hbm.at[pl.ds((b0 + c) * CHUNK, CHUNK)], buf, sem)

    def gout(c, buf, sem):
        return pltpu.make_async_copy(
            buf, out_hbm.at[pl.ds((b0 + c) * CHUNK, CHUNK)], sem)

    gin(0, in0, si0).start()
    gin(1, in1, si1).start()

    def process(c, ibuf, obuf, sin, sout):
        gin(c, ibuf, sin).wait()

        @pl.when(c >= 2)
        def _():
            gout(c - 2, obuf, sout).wait()

        @plsc.parallel_loop(0, 64, 1, unroll=2)
        def _t(t):
            nb = t >> 3
            g = t & 7
            src = pl.multiple_of(t * 16, 16)
            idxv = idv[pl.ds(src, 16)]
            obase = nb * 1024 + g * 16
            for ws in range(ROWS):
                v = plsc.load_gather(ibuf, [idxv + ws * 1024])
                dst = pl.multiple_of(
                    (ws // 8) * 8192 + (ws % 8) * 128 + obase, 16)
                obuf[pl.ds(dst, 16)] = v * SCALE

        gout(c, obuf, sout).start()

        @pl.when(c + 2 < BLK_PER_W)
        def _():
            gin(c + 2, ibuf, sin).start()

    def outer(t2, carry):
        process(2 * t2, in0, ob0, si0, so0)
        process(2 * t2 + 1, in1, ob1, si1, so1)
        return carry

    lax.fori_loop(0, BLK_PER_W // 2, outer, 0)
    gout(BLK_PER_W - 2, ob0, so0).wait()
    gout(BLK_PER_W - 1, ob1, so1).wait()


@jax.jit
def _sampler(flat_in, ids):
    mesh = plsc.VectorSubcoreMesh(core_axis_name="c", subcore_axis_name="s")
    run = functools.partial(
        pl.kernel,
        out_type=jax.ShapeDtypeStruct((TOTAL,), jnp.float32),
        mesh=mesh,
        compiler_params=pltpu.CompilerParams(needs_layout_passes=False),
        scratch_types=[
            pltpu.VMEM((N,), jnp.int32),
            pltpu.VMEM((CHUNK,), jnp.float32),
            pltpu.VMEM((CHUNK,), jnp.float32),
            pltpu.VMEM((CHUNK,), jnp.float32),
            pltpu.VMEM((CHUNK,), jnp.float32),
            pltpu.SemaphoreType.DMA,
            pltpu.SemaphoreType.DMA,
            pltpu.SemaphoreType.DMA,
            pltpu.SemaphoreType.DMA,
        ],
    )(_sampler_body)
    return run(flat_in, ids)


def kernel(masks, rand_id):
    flat_in = jnp.transpose(masks, (1, 2, 3, 0)).reshape(TOTAL)
    ids = rand_id.astype(jnp.int32)
    out1d = _sampler(flat_in, ids)
    out5 = out1d.reshape(H, W // 8, 8, 8, 128)
    out = jnp.transpose(out5, (2, 4, 0, 1, 3)).reshape(N, H, W)
    return out[:, None, :, :]
